# baseline (speedup 1.0000x reference)
import functools

import jax
import jax.numpy as jnp
from jax import lax
from jax.experimental import pallas as pl
from jax.experimental.pallas import tpu as pltpu
from jax.experimental.pallas import tpu_sc as plsc

_N = 10000
_E = 640000
_G = 64

_CH = 64
_NC = 2
_NS = 16
_NW = _NC * _NS
_CHUNKS = 320
_EPAD = _NW * _CH * _CHUNKS
_NPAD = 10112
_RPT = _NPAD // _NS


def _sc_segment_sum(table, edges3, zeros, d):
    mesh = plsc.VectorSubcoreMesh(core_axis_name="c", subcore_axis_name="s")
    nch = _CHUNKS

    @functools.partial(
        pl.kernel,
        out_type=jax.ShapeDtypeStruct((_NC, _NPAD, d), jnp.float32),
        mesh=mesh,
        compiler_params=pltpu.CompilerParams(use_tc_tiling_on_sc=False),
        scratch_types=[
            pltpu.VMEM((4, 2, _CH), jnp.int32),
            pltpu.VMEM((_CH, d), jnp.float32),
            pltpu.VMEM((_CH, d), jnp.float32),
            pltpu.VMEM((_CH, d), jnp.float32),
            pltpu.VMEM((_CH, d), jnp.float32),
            pltpu.VMEM_SHARED((_NPAD, d), jnp.float32),
            pltpu.SemaphoreType.DMA,
            pltpu.SemaphoreType.DMA,
            pltpu.SemaphoreType.DMA,
        ],
    )
    def k(table_h, edges_h, zeros_h, out_h,
          idx_v, db0, db1, db2, db3, acc_s, isem, gsem, ssem):
        dbufs = (db0, db1, db2, db3)
        cid = lax.axis_index("c")
        sid = lax.axis_index("s")
        wid = sid * _NC + cid
        row0 = wid * nch
        r0 = pl.multiple_of(sid * _RPT, 8)

        def idx(g, r):
            return pltpu.make_async_copy(edges_h.at[row0 + g], idx_v.at[r], isem)

        def gather(g, r):
            return pltpu.make_async_copy(
                table_h.at[idx_v.at[r, 0]], dbufs[r], gsem)

        def scatter(g, r):
            return pltpu.make_async_copy(
                dbufs[r], acc_s.at[idx_v.at[r, 1]], ssem)

        pltpu.sync_copy(zeros_h.at[pl.ds(r0, _RPT), :],
                        acc_s.at[pl.ds(r0, _RPT), :])
        plsc.subcore_barrier()

        idx(0, 0).start()
        idx(1, 1).start()
        idx(0, 0).wait()
        gather(0, 0).start()

        def quad(it, _):
            for r in range(4):
                g = it * 4 + r
                gather(g, r).wait()
                scatter(g, r).start(add=True)

                @pl.when(g >= 1)
                def _drain_prev():
                    scatter(g - 1, (r - 1) % 4).wait()

                @pl.when(g + 1 < nch)
                def _next_gather():
                    idx(g + 1, (r + 1) % 4).wait()
                    gather(g + 1, (r + 1) % 4).start()

                @pl.when(g + 2 < nch)
                def _next_idx():
                    idx(g + 2, (r + 2) % 4).start()
            return 0
        lax.fori_loop(0, nch // 4, quad, 0)
        scatter(nch - 1, 3).wait()
        plsc.subcore_barrier()

        pltpu.sync_copy(acc_s.at[pl.ds(r0, _RPT), :],
                        out_h.at[cid, pl.ds(r0, _RPT), :])

    return k(table, edges3, zeros)


def _tc_layer1(x0p, parts, w_root, w_rel, b):
    nb = 1000
    grid = _N // nb

    def body(x_r, p_r, wr_r, wl_r, b_r, o_r):
        agg = p_r[0] + p_r[1]
        acc = jnp.dot(x_r[...], wr_r[...], preferred_element_type=jnp.float32)
        acc += jnp.dot(agg, wl_r[...], preferred_element_type=jnp.float32)
        o_r[...] = jnp.maximum(acc + b_r[...], 0.0)

    return pl.pallas_call(
        body,
        grid=(grid,),
        in_specs=[
            pl.BlockSpec((nb, 16), lambda i: (i, 0)),
            pl.BlockSpec((2, nb, 16), lambda i: (0, i, 0)),
            pl.BlockSpec((16, 128), lambda i: (0, 0)),
            pl.BlockSpec((16, 128), lambda i: (0, 0)),
            pl.BlockSpec((1, 128), lambda i: (0, 0)),
        ],
        out_specs=pl.BlockSpec((nb, 128), lambda i: (i, 0)),
        out_shape=jax.ShapeDtypeStruct((_N, 128), jnp.float32),
    )(x0p, parts, w_root, w_rel, b)


def _tc_layer2_head(x1, parts, batch3, expad, w_root, w_rel, b2,
                    wl1a, wl1b, bl1, wl2, bl2, wl3, bl3):
    nb = 1000
    grid = _N // nb

    def body(x1_r, p_r, bt_r, ex_r, wr_r, wl_r, b2_r,
             w1a_r, w1b_r, b1_r, w2_r, b2h_r, w3_r, b3_r, o_r, pool):
        i = pl.program_id(0)

        @pl.when(i == 0)
        def _init():
            pool[...] = jnp.full((_G, 256), -jnp.inf, jnp.float32)

        x1b = x1_r[...]
        agg = p_r[0] + p_r[1]
        x2 = jnp.dot(x1b, wr_r[...], preferred_element_type=jnp.float32)
        x2 += jnp.dot(agg, wl_r[...], preferred_element_type=jnp.float32)
        x2 = jnp.maximum(x2 + b2_r[...], 0.0)

        bt = bt_r[0]
        g0 = jnp.min(bt)
        g1 = jnp.max(bt)

        def upd(g, _):
            m = bt == g
            m1 = jnp.max(jnp.where(m, x1b, -jnp.inf), axis=0, keepdims=True)
            m2 = jnp.max(jnp.where(m, x2, -jnp.inf), axis=0, keepdims=True)
            row = jnp.concatenate([m1, m2], axis=1)
            pool[pl.ds(g, 1), :] = jnp.maximum(pool[pl.ds(g, 1), :], row)
            return 0
        lax.fori_loop(g0, g1 + 1, upd, 0)

        @pl.when(i == grid - 1)
        def _head():
            p = pool[...]
            p = jnp.where(jnp.isfinite(p), p, 0.0)
            h = jnp.dot(p, w1a_r[...], preferred_element_type=jnp.float32)
            h += jnp.dot(ex_r[...], w1b_r[...], preferred_element_type=jnp.float32)
            h = jnp.maximum(h + b1_r[...], 0.0)
            h = jnp.maximum(jnp.dot(h, w2_r[...], preferred_element_type=jnp.float32) + b2h_r[...], 0.0)
            z = jnp.dot(h, w3_r[...], preferred_element_type=jnp.float32) + b3_r[...]
            zm = z - jnp.max(z, axis=-1, keepdims=True)
            o_r[...] = zm - jnp.log(jnp.sum(jnp.exp(zm), axis=-1, keepdims=True))

    return pl.pallas_call(
        body,
        grid=(grid,),
        in_specs=[
            pl.BlockSpec((nb, 128), lambda i: (i, 0)),
            pl.BlockSpec((2, nb, 128), lambda i: (0, i, 0)),
            pl.BlockSpec((1, nb, 1), lambda i: (i, 0, 0)),
            pl.BlockSpec((_G, 16), lambda i: (0, 0)),
            pl.BlockSpec((128, 128), lambda i: (0, 0)),
            pl.BlockSpec((128, 128), lambda i: (0, 0)),
            pl.BlockSpec((1, 128), lambda i: (0, 0)),
            pl.BlockSpec((256, 64), lambda i: (0, 0)),
            pl.BlockSpec((16, 64), lambda i: (0, 0)),
            pl.BlockSpec((1, 64), lambda i: (0, 0)),
            pl.BlockSpec((64, 32), lambda i: (0, 0)),
            pl.BlockSpec((1, 32), lambda i: (0, 0)),
            pl.BlockSpec((32, 10), lambda i: (0, 0)),
            pl.BlockSpec((1, 10), lambda i: (0, 0)),
        ],
        out_specs=pl.BlockSpec((_G, 10), lambda i: (0, 0)),
        out_shape=jax.ShapeDtypeStruct((_G, 10), jnp.float32),
        scratch_shapes=[pltpu.VMEM((_G, 256), jnp.float32)],
    )(x1, parts, batch3, expad, w_root, w_rel, b2,
      wl1a, wl1b, bl1, wl2, bl2, wl3, bl3)


def kernel(x, edge_index, batch, exinfo, W1_root, W1_rel, b1,
           W2_root, W2_rel, b2, Wl1, bl1, Wl2, bl2, Wl3, bl3):
    src = edge_index[0]
    dst = edge_index[1]
    pad = _EPAD - _E
    srcp = jnp.concatenate([src, jnp.zeros((pad,), jnp.int32)]).reshape(-1, _CH)
    dstp = jnp.concatenate([dst, jnp.full((pad,), _N, jnp.int32)]).reshape(-1, _CH)
    edges3 = jnp.stack([srcp, dstp], axis=1)

    x0p = jnp.pad(x[:, 2:5], ((0, 0), (0, 13)))
    w1r = jnp.pad(W1_root, ((0, 13), (0, 0)))
    w1l = jnp.pad(W1_rel, ((0, 13), (0, 0)))

    z16 = jnp.zeros((_NPAD, 16), jnp.float32)
    z128 = jnp.zeros((_NPAD, 128), jnp.float32)

    agg0 = _sc_segment_sum(x0p, edges3, z16, 16)
    x1 = _tc_layer1(x0p, agg0, w1r, w1l, b1.reshape(1, 128))

    agg1 = _sc_segment_sum(x1, edges3, z128, 128)

    batch3 = batch.reshape(_N // 1000, 1000, 1)
    expad = jnp.pad(exinfo, ((0, 0), (0, 6)))
    wl1a = Wl1[:256]
    wl1b = jnp.pad(Wl1[256:], ((0, 6), (0, 0)))

    return _tc_layer2_head(
        x1, agg1, batch3, expad, W2_root, W2_rel, b2.reshape(1, 128),
        wl1a, wl1b, bl1.reshape(1, 64), Wl2, bl2.reshape(1, 32),
        Wl3, bl3.reshape(1, 10))


# speedup vs baseline: 2.9740x; 2.9740x over previous
import functools

import jax
import jax.numpy as jnp
from jax import lax
from jax.experimental import pallas as pl
from jax.experimental.pallas import tpu as pltpu
from jax.experimental.pallas import tpu_sc as plsc

_N = 10000
_E = 640000
_G = 64

_CH = 128
_NC = 2
_NS = 16
_NW = _NC * _NS
_CHUNKS = 160
_EPAD = _NW * _CH * _CHUNKS
_NPAD = 10112
_RPT = _NPAD // _NS
_RSTG = _N // _NS


def _sc_segment_sum(table, edges3, zeros, d, split_features):
    mesh = plsc.VectorSubcoreMesh(core_axis_name="c", subcore_axis_name="s")
    nch = _CHUNKS * (2 if split_features else 1)

    @functools.partial(
        pl.kernel,
        out_type=jax.ShapeDtypeStruct((_NC, _NPAD, d), jnp.float32),
        mesh=mesh,
        compiler_params=pltpu.CompilerParams(use_tc_tiling_on_sc=False),
        scratch_types=[
            pltpu.VMEM((4, 2, _CH), jnp.int32),
            pltpu.VMEM((_CH, d), jnp.float32),
            pltpu.VMEM((_CH, d), jnp.float32),
            pltpu.VMEM((_CH, d), jnp.float32),
            pltpu.VMEM((_CH, d), jnp.float32),
            pltpu.VMEM_SHARED((_NPAD, d), jnp.float32),
            pltpu.VMEM_SHARED((_N, d), jnp.float32),
            pltpu.SemaphoreType.DMA,
            pltpu.SemaphoreType.DMA,
            pltpu.SemaphoreType.DMA,
        ],
    )
    def k(table_h, edges_h, zeros_h, out_h,
          idx_v, db0, db1, db2, db3, acc_s, tab_s, isem, gsem, ssem):
        dbufs = (db0, db1, db2, db3)
        cid = lax.axis_index("c")
        sid = lax.axis_index("s")
        wid = sid * _NC + cid
        row0 = (sid if split_features else wid) * nch
        r0 = pl.multiple_of(sid * _RPT, 8)
        sr0 = sid * _RSTG

        def idx(g, r):
            return pltpu.make_async_copy(edges_h.at[row0 + g], idx_v.at[r], isem)

        def gather(g, r):
            return pltpu.make_async_copy(
                tab_s.at[idx_v.at[r, 0]], dbufs[r], gsem)

        def scatter(g, r):
            return pltpu.make_async_copy(
                dbufs[r], acc_s.at[idx_v.at[r, 1]], ssem)

        pltpu.sync_copy(zeros_h.at[pl.ds(r0, _RPT), :],
                        acc_s.at[pl.ds(r0, _RPT), :])
        if split_features:
            pltpu.sync_copy(table_h.at[cid, pl.ds(sr0, _RSTG), :],
                            tab_s.at[pl.ds(sr0, _RSTG), :])
        else:
            pltpu.sync_copy(table_h.at[pl.ds(sr0, _RSTG), :],
                            tab_s.at[pl.ds(sr0, _RSTG), :])
        plsc.subcore_barrier()

        idx(0, 0).start()
        idx(1, 1).start()
        idx(0, 0).wait()
        gather(0, 0).start()

        def quad(it, _):
            for r in range(4):
                g = it * 4 + r
                gather(g, r).wait()
                scatter(g, r).start(add=True)

                @pl.when(g >= 1)
                def _drain_prev():
                    scatter(g - 1, (r - 1) % 4).wait()

                @pl.when(g + 1 < nch)
                def _next_gather():
                    idx(g + 1, (r + 1) % 4).wait()
                    gather(g + 1, (r + 1) % 4).start()

                @pl.when(g + 2 < nch)
                def _next_idx():
                    idx(g + 2, (r + 2) % 4).start()
            return 0
        lax.fori_loop(0, nch // 4, quad, 0)
        scatter(nch - 1, 3).wait()
        plsc.subcore_barrier()

        pltpu.sync_copy(acc_s.at[pl.ds(r0, _RPT), :],
                        out_h.at[cid, pl.ds(r0, _RPT), :])

    return k(table, edges3, zeros)


def _tc_layer1(x0p, parts, w_root, w_rel, b):
    nb = 1000
    grid = _N // nb

    def body(x_r, p_r, wr_r, wl_r, b_r, o_r):
        agg = p_r[0] + p_r[1]
        acc = jnp.dot(x_r[...], wr_r[...], preferred_element_type=jnp.float32)
        acc += jnp.dot(agg, wl_r[...], preferred_element_type=jnp.float32)
        x1 = jnp.maximum(acc + b_r[...], 0.0)
        o_r[0] = x1[:, :64]
        o_r[1] = x1[:, 64:]

    return pl.pallas_call(
        body,
        grid=(grid,),
        in_specs=[
            pl.BlockSpec((nb, 16), lambda i: (i, 0)),
            pl.BlockSpec((2, nb, 16), lambda i: (0, i, 0)),
            pl.BlockSpec((16, 128), lambda i: (0, 0)),
            pl.BlockSpec((16, 128), lambda i: (0, 0)),
            pl.BlockSpec((1, 128), lambda i: (0, 0)),
        ],
        out_specs=pl.BlockSpec((2, nb, 64), lambda i: (0, i, 0)),
        out_shape=jax.ShapeDtypeStruct((2, _N, 64), jnp.float32),
    )(x0p, parts, w_root, w_rel, b)


def _tc_layer2_head(x1, parts, batch3, expad, w_root, w_rel, b2,
                    wl1a, wl1b, bl1, wl2, bl2, wl3, bl3):
    nb = 1000
    grid = _N // nb

    def body(x1_r, p_r, bt_r, ex_r, wr_r, wl_r, b2_r,
             w1a_r, w1b_r, b1_r, w2_r, b2h_r, w3_r, b3_r, o_r, pool):
        i = pl.program_id(0)

        @pl.when(i == 0)
        def _init():
            pool[...] = jnp.full((_G, 256), -jnp.inf, jnp.float32)

        x1b = jnp.concatenate([x1_r[0], x1_r[1]], axis=1)
        agg = jnp.concatenate([p_r[0], p_r[1]], axis=1)
        x2 = jnp.dot(x1b, wr_r[...], preferred_element_type=jnp.float32)
        x2 += jnp.dot(agg, wl_r[...], preferred_element_type=jnp.float32)
        x2 = jnp.maximum(x2 + b2_r[...], 0.0)

        bt = bt_r[0]
        g0 = jnp.min(bt)
        g1 = jnp.max(bt)

        def upd(g, _):
            m = bt == g
            m1 = jnp.max(jnp.where(m, x1b, -jnp.inf), axis=0, keepdims=True)
            m2 = jnp.max(jnp.where(m, x2, -jnp.inf), axis=0, keepdims=True)
            row = jnp.concatenate([m1, m2], axis=1)
            pool[pl.ds(g, 1), :] = jnp.maximum(pool[pl.ds(g, 1), :], row)
            return 0
        lax.fori_loop(g0, g1 + 1, upd, 0)

        @pl.when(i == grid - 1)
        def _head():
            p = pool[...]
            p = jnp.where(jnp.isfinite(p), p, 0.0)
            h = jnp.dot(p, w1a_r[...], preferred_element_type=jnp.float32)
            h += jnp.dot(ex_r[...], w1b_r[...], preferred_element_type=jnp.float32)
            h = jnp.maximum(h + b1_r[...], 0.0)
            h = jnp.maximum(jnp.dot(h, w2_r[...], preferred_element_type=jnp.float32) + b2h_r[...], 0.0)
            z = jnp.dot(h, w3_r[...], preferred_element_type=jnp.float32) + b3_r[...]
            zm = z - jnp.max(z, axis=-1, keepdims=True)
            o_r[...] = zm - jnp.log(jnp.sum(jnp.exp(zm), axis=-1, keepdims=True))

    return pl.pallas_call(
        body,
        grid=(grid,),
        in_specs=[
            pl.BlockSpec((2, nb, 64), lambda i: (0, i, 0)),
            pl.BlockSpec((2, nb, 64), lambda i: (0, i, 0)),
            pl.BlockSpec((1, nb, 1), lambda i: (i, 0, 0)),
            pl.BlockSpec((_G, 16), lambda i: (0, 0)),
            pl.BlockSpec((128, 128), lambda i: (0, 0)),
            pl.BlockSpec((128, 128), lambda i: (0, 0)),
            pl.BlockSpec((1, 128), lambda i: (0, 0)),
            pl.BlockSpec((256, 64), lambda i: (0, 0)),
            pl.BlockSpec((16, 64), lambda i: (0, 0)),
            pl.BlockSpec((1, 64), lambda i: (0, 0)),
            pl.BlockSpec((64, 32), lambda i: (0, 0)),
            pl.BlockSpec((1, 32), lambda i: (0, 0)),
            pl.BlockSpec((32, 10), lambda i: (0, 0)),
            pl.BlockSpec((1, 10), lambda i: (0, 0)),
        ],
        out_specs=pl.BlockSpec((_G, 10), lambda i: (0, 0)),
        out_shape=jax.ShapeDtypeStruct((_G, 10), jnp.float32),
        scratch_shapes=[pltpu.VMEM((_G, 256), jnp.float32)],
    )(x1, parts, batch3, expad, w_root, w_rel, b2,
      wl1a, wl1b, bl1, wl2, bl2, wl3, bl3)


def kernel(x, edge_index, batch, exinfo, W1_root, W1_rel, b1,
           W2_root, W2_rel, b2, Wl1, bl1, Wl2, bl2, Wl3, bl3):
    src = edge_index[0]
    dst = edge_index[1]
    pad = _EPAD - _E
    srcp = jnp.concatenate([src, jnp.zeros((pad,), jnp.int32)]).reshape(-1, _CH)
    dstp = jnp.concatenate([dst, jnp.full((pad,), _N, jnp.int32)]).reshape(-1, _CH)
    edges3 = jnp.stack([srcp, dstp], axis=1)

    x0p = jnp.pad(x[:, 2:5], ((0, 0), (0, 13)))
    w1r = jnp.pad(W1_root, ((0, 13), (0, 0)))
    w1l = jnp.pad(W1_rel, ((0, 13), (0, 0)))

    z16 = jnp.zeros((_NPAD, 16), jnp.float32)
    z64 = jnp.zeros((_NPAD, 64), jnp.float32)

    agg0 = _sc_segment_sum(x0p, edges3, z16, 16, False)
    x1 = _tc_layer1(x0p, agg0, w1r, w1l, b1.reshape(1, 128))

    agg1 = _sc_segment_sum(x1, edges3, z64, 64, True)

    batch3 = batch.reshape(_N // 1000, 1000, 1)
    expad = jnp.pad(exinfo, ((0, 0), (0, 6)))
    wl1a = Wl1[:256]
    wl1b = jnp.pad(Wl1[256:], ((0, 6), (0, 0)))

    return _tc_layer2_head(
        x1, agg1, batch3, expad, W2_root, W2_rel, b2.reshape(1, 128),
        wl1a, wl1b, bl1.reshape(1, 64), Wl2, bl2.reshape(1, 32),
        Wl3, bl3.reshape(1, 10))


# speedup vs baseline: 3.2843x; 1.1043x over previous
import functools

import jax
import jax.numpy as jnp
from jax import lax
from jax.experimental import pallas as pl
from jax.experimental.pallas import tpu as pltpu
from jax.experimental.pallas import tpu_sc as plsc

_N = 10000
_E = 640000
_G = 64

_CH = 128
_NC = 2
_NS = 16
_NW = _NC * _NS
_CHUNKS = 160
_EPAD = _NW * _CH * _CHUNKS
_NPAD = 10112
_RPT = _NPAD // _NS
_RSTG = _N // _NS


def _sc_segment_sum(table, edges3, zeros, d, split_features):
    mesh = plsc.VectorSubcoreMesh(core_axis_name="c", subcore_axis_name="s")
    nch = _CHUNKS * (2 if split_features else 1)
    ns, gp, sl, ip = (5, 2, 2, 3) if split_features else (8, 3, 4, 4)

    @functools.partial(
        pl.kernel,
        out_type=jax.ShapeDtypeStruct((_NC, _NPAD, d), jnp.float32),
        mesh=mesh,
        compiler_params=pltpu.CompilerParams(use_tc_tiling_on_sc=False),
        scratch_types=[
            pltpu.VMEM((ns, 2, _CH), jnp.int32),
        ] + [pltpu.VMEM((_CH, d), jnp.float32) for _ in range(ns)] + [
            pltpu.VMEM_SHARED((_NPAD, d), jnp.float32),
            pltpu.VMEM_SHARED((_N, d), jnp.float32),
            pltpu.SemaphoreType.DMA,
            pltpu.SemaphoreType.DMA,
            pltpu.SemaphoreType.DMA,
        ],
    )
    def k(table_h, edges_h, zeros_h, out_h, idx_v, *rest):
        dbufs = rest[:ns]
        acc_s, tab_s, isem, gsem, ssem = rest[ns:]
        cid = lax.axis_index("c")
        sid = lax.axis_index("s")
        wid = sid * _NC + cid
        row0 = (sid if split_features else wid) * nch
        r0 = pl.multiple_of(sid * _RPT, 8)
        sr0 = sid * _RSTG

        def idx(g, r):
            return pltpu.make_async_copy(edges_h.at[row0 + g], idx_v.at[r], isem)

        def gather(g, r):
            return pltpu.make_async_copy(
                tab_s.at[idx_v.at[r, 0]], dbufs[r], gsem)

        def scatter(g, r):
            return pltpu.make_async_copy(
                dbufs[r], acc_s.at[idx_v.at[r, 1]], ssem)

        pltpu.sync_copy(zeros_h.at[pl.ds(r0, _RPT), :],
                        acc_s.at[pl.ds(r0, _RPT), :])
        if split_features:
            pltpu.sync_copy(table_h.at[cid, pl.ds(sr0, _RSTG), :],
                            tab_s.at[pl.ds(sr0, _RSTG), :])
        else:
            pltpu.sync_copy(table_h.at[pl.ds(sr0, _RSTG), :],
                            tab_s.at[pl.ds(sr0, _RSTG), :])
        plsc.subcore_barrier()

        for j in range(ip):
            idx(j, j).start()
        for j in range(gp):
            idx(j, j).wait()
            gather(j, j).start()

        def ring(it, _):
            for r in range(ns):
                g = it * ns + r
                gather(g, r).wait()
                scatter(g, r).start(add=True)

                @pl.when(g >= sl)
                def _drain_scatter():
                    scatter(g - sl, (r - sl) % ns).wait()

                @pl.when(g + gp < nch)
                def _next_gather():
                    idx(g + gp, (r + gp) % ns).wait()
                    gather(g + gp, (r + gp) % ns).start()

                @pl.when(g + ip < nch)
                def _next_idx():
                    idx(g + ip, (r + ip) % ns).start()
            return 0
        lax.fori_loop(0, nch // ns, ring, 0)
        for t in range(sl):
            g = nch - sl + t
            scatter(g, g % ns).wait()
        plsc.subcore_barrier()

        pltpu.sync_copy(acc_s.at[pl.ds(r0, _RPT), :],
                        out_h.at[cid, pl.ds(r0, _RPT), :])

    return k(table, edges3, zeros)


def _tc_layer1(x0p, parts, w_root, w_rel, b):
    nb = 1000
    grid = _N // nb

    def body(x_r, p_r, wr_r, wl_r, b_r, o_r):
        agg = p_r[0] + p_r[1]
        acc = jnp.dot(x_r[...], wr_r[...], preferred_element_type=jnp.float32)
        acc += jnp.dot(agg, wl_r[...], preferred_element_type=jnp.float32)
        x1 = jnp.maximum(acc + b_r[...], 0.0)
        o_r[0] = x1[:, :64]
        o_r[1] = x1[:, 64:]

    return pl.pallas_call(
        body,
        grid=(grid,),
        in_specs=[
            pl.BlockSpec((nb, 8), lambda i: (i, 0)),
            pl.BlockSpec((2, nb, 8), lambda i: (0, i, 0)),
            pl.BlockSpec((8, 128), lambda i: (0, 0)),
            pl.BlockSpec((8, 128), lambda i: (0, 0)),
            pl.BlockSpec((1, 128), lambda i: (0, 0)),
        ],
        out_specs=pl.BlockSpec((2, nb, 64), lambda i: (0, i, 0)),
        out_shape=jax.ShapeDtypeStruct((2, _N, 64), jnp.float32),
    )(x0p, parts, w_root, w_rel, b)


def _tc_layer2_head(x1, parts, batch3, expad, w_root, w_rel, b2,
                    wl1a, wl1b, bl1, wl2, bl2, wl3, bl3):
    nb = 1000
    grid = _N // nb

    def body(x1_r, p_r, bt_r, ex_r, wr_r, wl_r, b2_r,
             w1a_r, w1b_r, b1_r, w2_r, b2h_r, w3_r, b3_r, o_r, pool):
        i = pl.program_id(0)

        @pl.when(i == 0)
        def _init():
            pool[...] = jnp.full((_G, 256), -jnp.inf, jnp.float32)

        x1b = jnp.concatenate([x1_r[0], x1_r[1]], axis=1)
        agg = jnp.concatenate([p_r[0], p_r[1]], axis=1)
        x2 = jnp.dot(x1b, wr_r[...], preferred_element_type=jnp.float32)
        x2 += jnp.dot(agg, wl_r[...], preferred_element_type=jnp.float32)
        x2 = jnp.maximum(x2 + b2_r[...], 0.0)

        bt = bt_r[0]
        g0 = jnp.min(bt)
        g1 = jnp.max(bt)

        def upd(g, _):
            m = bt == g
            m1 = jnp.max(jnp.where(m, x1b, -jnp.inf), axis=0, keepdims=True)
            m2 = jnp.max(jnp.where(m, x2, -jnp.inf), axis=0, keepdims=True)
            row = jnp.concatenate([m1, m2], axis=1)
            pool[pl.ds(g, 1), :] = jnp.maximum(pool[pl.ds(g, 1), :], row)
            return 0
        lax.fori_loop(g0, g1 + 1, upd, 0)

        @pl.when(i == grid - 1)
        def _head():
            p = pool[...]
            p = jnp.where(jnp.isfinite(p), p, 0.0)
            h = jnp.dot(p, w1a_r[...], preferred_element_type=jnp.float32)
            h += jnp.dot(ex_r[...], w1b_r[...], preferred_element_type=jnp.float32)
            h = jnp.maximum(h + b1_r[...], 0.0)
            h = jnp.maximum(jnp.dot(h, w2_r[...], preferred_element_type=jnp.float32) + b2h_r[...], 0.0)
            z = jnp.dot(h, w3_r[...], preferred_element_type=jnp.float32) + b3_r[...]
            zm = z - jnp.max(z, axis=-1, keepdims=True)
            o_r[...] = zm - jnp.log(jnp.sum(jnp.exp(zm), axis=-1, keepdims=True))

    return pl.pallas_call(
        body,
        grid=(grid,),
        in_specs=[
            pl.BlockSpec((2, nb, 64), lambda i: (0, i, 0)),
            pl.BlockSpec((2, nb, 64), lambda i: (0, i, 0)),
            pl.BlockSpec((1, nb, 1), lambda i: (i, 0, 0)),
            pl.BlockSpec((_G, 16), lambda i: (0, 0)),
            pl.BlockSpec((128, 128), lambda i: (0, 0)),
            pl.BlockSpec((128, 128), lambda i: (0, 0)),
            pl.BlockSpec((1, 128), lambda i: (0, 0)),
            pl.BlockSpec((256, 64), lambda i: (0, 0)),
            pl.BlockSpec((16, 64), lambda i: (0, 0)),
            pl.BlockSpec((1, 64), lambda i: (0, 0)),
            pl.BlockSpec((64, 32), lambda i: (0, 0)),
            pl.BlockSpec((1, 32), lambda i: (0, 0)),
            pl.BlockSpec((32, 10), lambda i: (0, 0)),
            pl.BlockSpec((1, 10), lambda i: (0, 0)),
        ],
        out_specs=pl.BlockSpec((_G, 10), lambda i: (0, 0)),
        out_shape=jax.ShapeDtypeStruct((_G, 10), jnp.float32),
        scratch_shapes=[pltpu.VMEM((_G, 256), jnp.float32)],
    )(x1, parts, batch3, expad, w_root, w_rel, b2,
      wl1a, wl1b, bl1, wl2, bl2, wl3, bl3)


def kernel(x, edge_index, batch, exinfo, W1_root, W1_rel, b1,
           W2_root, W2_rel, b2, Wl1, bl1, Wl2, bl2, Wl3, bl3):
    src = edge_index[0]
    dst = edge_index[1]
    pad = _EPAD - _E
    srcp = jnp.concatenate([src, jnp.zeros((pad,), jnp.int32)]).reshape(-1, _CH)
    dstp = jnp.concatenate([dst, jnp.full((pad,), _N, jnp.int32)]).reshape(-1, _CH)
    edges3 = jnp.stack([srcp, dstp], axis=1)

    x0p = jnp.pad(x[:, 2:5], ((0, 0), (0, 5)))
    w1r = jnp.pad(W1_root, ((0, 5), (0, 0)))
    w1l = jnp.pad(W1_rel, ((0, 5), (0, 0)))

    z8 = jnp.zeros((_NPAD, 8), jnp.float32)
    z64 = jnp.zeros((_NPAD, 64), jnp.float32)

    agg0 = _sc_segment_sum(x0p, edges3, z8, 8, False)
    x1 = _tc_layer1(x0p, agg0, w1r, w1l, b1.reshape(1, 128))

    agg1 = _sc_segment_sum(x1, edges3, z64, 64, True)

    batch3 = batch.reshape(_N // 1000, 1000, 1)
    expad = jnp.pad(exinfo, ((0, 0), (0, 6)))
    wl1a = Wl1[:256]
    wl1b = jnp.pad(Wl1[256:], ((0, 6), (0, 0)))

    return _tc_layer2_head(
        x1, agg1, batch3, expad, W2_root, W2_rel, b2.reshape(1, 128),
        wl1a, wl1b, bl1.reshape(1, 64), Wl2, bl2.reshape(1, 32),
        Wl3, bl3.reshape(1, 10))


# speedup vs baseline: 3.7344x; 1.1371x over previous
import functools

import jax
import jax.numpy as jnp
from jax import lax
from jax.experimental import pallas as pl
from jax.experimental.pallas import tpu as pltpu
from jax.experimental.pallas import tpu_sc as plsc

_N = 10000
_E = 640000
_G = 64

_CH = 128
_NC = 2
_NS = 16
_NW = _NC * _NS
_CHUNKS = 160
_EPAD = _NW * _CH * _CHUNKS
_NPAD = 10112
_RPT = _NPAD // _NS
_RSTG = _N // _NS


def _sc_segment_sum(table, edges3, zeros, d, split_features):
    mesh = plsc.VectorSubcoreMesh(core_axis_name="c", subcore_axis_name="s")
    nch = _CHUNKS * (2 if split_features else 1)
    ns, gp, sl = (5, 2, 2) if split_features else (8, 3, 4)
    bs = 2 * ns
    nblk = nch // bs

    @functools.partial(
        pl.kernel,
        out_type=jax.ShapeDtypeStruct((_NC, _NPAD, d), jnp.float32),
        mesh=mesh,
        compiler_params=pltpu.CompilerParams(use_tc_tiling_on_sc=False),
        scratch_types=[
            pltpu.VMEM((bs, 2, _CH), jnp.int32),
            pltpu.VMEM((bs, 2, _CH), jnp.int32),
        ] + [pltpu.VMEM((_CH, d), jnp.float32) for _ in range(ns)] + [
            pltpu.VMEM_SHARED((_NPAD, d), jnp.float32),
            pltpu.VMEM_SHARED((_N, d), jnp.float32),
            pltpu.SemaphoreType.DMA,
            pltpu.SemaphoreType.DMA,
            pltpu.SemaphoreType.DMA,
        ],
    )
    def k(table_h, edges_h, zeros_h, out_h, ib0, ib1, *rest):
        ibufs = (ib0, ib1)
        dbufs = rest[:ns]
        acc_s, tab_s, isem, gsem, ssem = rest[ns:]
        cid = lax.axis_index("c")
        sid = lax.axis_index("s")
        wid = sid * _NC + cid
        row0 = (sid if split_features else wid) * nch
        r0 = pl.multiple_of(sid * _RPT, 8)
        sr0 = sid * _RSTG

        def idxblk(b, h):
            return pltpu.make_async_copy(
                edges_h.at[pl.ds(row0 + b * bs, bs), :, :], ibufs[h], isem)

        def irow(half, q, which):
            hh, qq = (half + q // bs) % 2, q % bs
            return ibufs[hh].at[qq, which]

        def gather(half, q, g):
            return pltpu.make_async_copy(
                tab_s.at[irow(half, q, 0)], dbufs[(g) % ns], gsem)

        def scatter(half, q, g):
            return pltpu.make_async_copy(
                dbufs[(g) % ns], acc_s.at[irow(half, q, 1)], ssem)

        pltpu.sync_copy(zeros_h.at[pl.ds(r0, _RPT), :],
                        acc_s.at[pl.ds(r0, _RPT), :])
        if split_features:
            pltpu.sync_copy(table_h.at[cid, pl.ds(sr0, _RSTG), :],
                            tab_s.at[pl.ds(sr0, _RSTG), :])
        else:
            pltpu.sync_copy(table_h.at[pl.ds(sr0, _RSTG), :],
                            tab_s.at[pl.ds(sr0, _RSTG), :])
        plsc.subcore_barrier()

        idxblk(0, 0).start()
        idxblk(0, 0).wait()
        for j in range(gp):
            gather(0, j, j).start()

        def body(it, _):
            for half in range(2):
                b = it * 2 + half
                for q in range(bs):
                    g = b * bs + q
                    gather(half, q, q).wait()
                    scatter(half, q, q).start(add=True)

                    @pl.when(g >= sl)
                    def _drain_scatter():
                        scatter(half, q - sl, q - sl).wait()

                    if q == sl:
                        @pl.when(b + 1 < nblk)
                        def _next_idxblk():
                            idxblk(b + 1, 1 - half).start()

                    if q == bs - gp:
                        @pl.when(b + 1 < nblk)
                        def _wait_idxblk():
                            idxblk(b + 1, 1 - half).wait()

                    @pl.when(g + gp < nch)
                    def _next_gather():
                        gather(half, q + gp, q + gp).start()
            return 0
        lax.fori_loop(0, nch // (2 * bs), body, 0)
        for t in range(sl):
            q = bs - sl + t
            scatter(1, q, q).wait()
        plsc.subcore_barrier()

        pltpu.sync_copy(acc_s.at[pl.ds(r0, _RPT), :],
                        out_h.at[cid, pl.ds(r0, _RPT), :])

    return k(table, edges3, zeros)


def _tc_layer1(x0p, parts, w_root, w_rel, b):
    nb = 1000
    grid = _N // nb

    def body(x_r, p_r, wr_r, wl_r, b_r, o_r):
        agg = p_r[0] + p_r[1]
        acc = jnp.dot(x_r[...], wr_r[...], preferred_element_type=jnp.float32)
        acc += jnp.dot(agg, wl_r[...], preferred_element_type=jnp.float32)
        x1 = jnp.maximum(acc + b_r[...], 0.0)
        o_r[0] = x1[:, :64]
        o_r[1] = x1[:, 64:]

    return pl.pallas_call(
        body,
        grid=(grid,),
        in_specs=[
            pl.BlockSpec((nb, 8), lambda i: (i, 0)),
            pl.BlockSpec((2, nb, 8), lambda i: (0, i, 0)),
            pl.BlockSpec((8, 128), lambda i: (0, 0)),
            pl.BlockSpec((8, 128), lambda i: (0, 0)),
            pl.BlockSpec((1, 128), lambda i: (0, 0)),
        ],
        out_specs=pl.BlockSpec((2, nb, 64), lambda i: (0, i, 0)),
        out_shape=jax.ShapeDtypeStruct((2, _N, 64), jnp.float32),
    )(x0p, parts, w_root, w_rel, b)


def _tc_layer2_head(x1, parts, batch3, expad, w_root, w_rel, b2,
                    wl1a, wl1b, bl1, wl2, bl2, wl3, bl3):
    nb = 1000
    grid = _N // nb

    def body(x1_r, p_r, bt_r, ex_r, wr_r, wl_r, b2_r,
             w1a_r, w1b_r, b1_r, w2_r, b2h_r, w3_r, b3_r, o_r, pool):
        i = pl.program_id(0)

        @pl.when(i == 0)
        def _init():
            pool[...] = jnp.full((_G, 256), -jnp.inf, jnp.float32)

        x1b = jnp.concatenate([x1_r[0], x1_r[1]], axis=1)
        agg = jnp.concatenate([p_r[0], p_r[1]], axis=1)
        x2 = jnp.dot(x1b, wr_r[...], preferred_element_type=jnp.float32)
        x2 += jnp.dot(agg, wl_r[...], preferred_element_type=jnp.float32)
        x2 = jnp.maximum(x2 + b2_r[...], 0.0)

        bt = bt_r[0]
        g0 = jnp.min(bt)
        g1 = jnp.max(bt)

        def upd(g, _):
            m = bt == g
            m1 = jnp.max(jnp.where(m, x1b, -jnp.inf), axis=0, keepdims=True)
            m2 = jnp.max(jnp.where(m, x2, -jnp.inf), axis=0, keepdims=True)
            row = jnp.concatenate([m1, m2], axis=1)
            pool[pl.ds(g, 1), :] = jnp.maximum(pool[pl.ds(g, 1), :], row)
            return 0
        lax.fori_loop(g0, g1 + 1, upd, 0)

        @pl.when(i == grid - 1)
        def _head():
            p = pool[...]
            p = jnp.where(jnp.isfinite(p), p, 0.0)
            h = jnp.dot(p, w1a_r[...], preferred_element_type=jnp.float32)
            h += jnp.dot(ex_r[...], w1b_r[...], preferred_element_type=jnp.float32)
            h = jnp.maximum(h + b1_r[...], 0.0)
            h = jnp.maximum(jnp.dot(h, w2_r[...], preferred_element_type=jnp.float32) + b2h_r[...], 0.0)
            z = jnp.dot(h, w3_r[...], preferred_element_type=jnp.float32) + b3_r[...]
            zm = z - jnp.max(z, axis=-1, keepdims=True)
            o_r[...] = zm - jnp.log(jnp.sum(jnp.exp(zm), axis=-1, keepdims=True))

    return pl.pallas_call(
        body,
        grid=(grid,),
        in_specs=[
            pl.BlockSpec((2, nb, 64), lambda i: (0, i, 0)),
            pl.BlockSpec((2, nb, 64), lambda i: (0, i, 0)),
            pl.BlockSpec((1, nb, 1), lambda i: (i, 0, 0)),
            pl.BlockSpec((_G, 16), lambda i: (0, 0)),
            pl.BlockSpec((128, 128), lambda i: (0, 0)),
            pl.BlockSpec((128, 128), lambda i: (0, 0)),
            pl.BlockSpec((1, 128), lambda i: (0, 0)),
            pl.BlockSpec((256, 64), lambda i: (0, 0)),
            pl.BlockSpec((16, 64), lambda i: (0, 0)),
            pl.BlockSpec((1, 64), lambda i: (0, 0)),
            pl.BlockSpec((64, 32), lambda i: (0, 0)),
            pl.BlockSpec((1, 32), lambda i: (0, 0)),
            pl.BlockSpec((32, 10), lambda i: (0, 0)),
            pl.BlockSpec((1, 10), lambda i: (0, 0)),
        ],
        out_specs=pl.BlockSpec((_G, 10), lambda i: (0, 0)),
        out_shape=jax.ShapeDtypeStruct((_G, 10), jnp.float32),
        scratch_shapes=[pltpu.VMEM((_G, 256), jnp.float32)],
    )(x1, parts, batch3, expad, w_root, w_rel, b2,
      wl1a, wl1b, bl1, wl2, bl2, wl3, bl3)


def kernel(x, edge_index, batch, exinfo, W1_root, W1_rel, b1,
           W2_root, W2_rel, b2, Wl1, bl1, Wl2, bl2, Wl3, bl3):
    src = edge_index[0]
    dst = edge_index[1]
    pad = _EPAD - _E
    srcp = jnp.concatenate([src, jnp.zeros((pad,), jnp.int32)]).reshape(-1, _CH)
    dstp = jnp.concatenate([dst, jnp.full((pad,), _N, jnp.int32)]).reshape(-1, _CH)
    edges3 = jnp.stack([srcp, dstp], axis=1)

    x0p = jnp.pad(x[:, 2:5], ((0, 0), (0, 5)))
    w1r = jnp.pad(W1_root, ((0, 5), (0, 0)))
    w1l = jnp.pad(W1_rel, ((0, 5), (0, 0)))

    z8 = jnp.zeros((_NPAD, 8), jnp.float32)
    z64 = jnp.zeros((_NPAD, 64), jnp.float32)

    agg0 = _sc_segment_sum(x0p, edges3, z8, 8, False)
    x1 = _tc_layer1(x0p, agg0, w1r, w1l, b1.reshape(1, 128))

    agg1 = _sc_segment_sum(x1, edges3, z64, 64, True)

    batch3 = batch.reshape(_N // 1000, 1000, 1)
    expad = jnp.pad(exinfo, ((0, 0), (0, 6)))
    wl1a = Wl1[:256]
    wl1b = jnp.pad(Wl1[256:], ((0, 6), (0, 0)))

    return _tc_layer2_head(
        x1, agg1, batch3, expad, W2_root, W2_rel, b2.reshape(1, 128),
        wl1a, wl1b, bl1.reshape(1, 64), Wl2, bl2.reshape(1, 32),
        Wl3, bl3.reshape(1, 10))
